# baseline (speedup 1.0000x reference)
import jax
import jax.numpy as jnp
from jax import lax
from jax.experimental import pallas as pl
from jax.experimental.pallas import tpu as pltpu
from jax.experimental.pallas import tpu_sc as plsc

_B = 16384
_D = 64
_NW = 32
_BPW = _B // _NW
_ICH = 128
_NCH = _BPW // _ICH
_GROUPS = _BPW // 16


def _body(uidx_hbm, iidx_hbm, utab_hbm, itab_hbm, w_hbm, b_hbm, out_hbm,
          uidx_v, iidx_v, urows_v, irows_v, w_v, b_v, out_v, sem):
    wid = lax.axis_index("s") * 2 + lax.axis_index("c")
    base = wid * _BPW

    for k in range(_NCH):
        pltpu.sync_copy(uidx_hbm.at[pl.ds(base + k * _ICH, _ICH)],
                        uidx_v.at[k])
        pltpu.sync_copy(iidx_hbm.at[pl.ds(base + k * _ICH, _ICH)],
                        iidx_v.at[k])
    pltpu.sync_copy(w_hbm, w_v)
    pltpu.sync_copy(b_hbm, b_v)

    copies = []
    for k in range(_NCH):
        copies.append(pltpu.async_copy(
            utab_hbm.at[uidx_v.at[k]],
            urows_v.at[pl.ds(k * _ICH, _ICH)], sem))
        copies.append(pltpu.async_copy(
            itab_hbm.at[iidx_v.at[k]],
            irows_v.at[pl.ds(k * _ICH, _ICH)], sem))
    for c in copies:
        c.wait()

    wv = [w_v[pl.ds(c * 16, 16)] for c in range(8)]
    bv = b_v[...]
    lane = lax.broadcasted_iota(jnp.int32, (16,), 0)

    def group(g, carry):
        acc = jnp.zeros((16,), jnp.float32)
        for rr in range(16):
            r = g * 16 + rr
            p = urows_v[r, pl.ds(0, 16)] * wv[0]
            p = p + urows_v[r, pl.ds(16, 16)] * wv[1]
            p = p + urows_v[r, pl.ds(32, 16)] * wv[2]
            p = p + urows_v[r, pl.ds(48, 16)] * wv[3]
            p = p + irows_v[r, pl.ds(0, 16)] * wv[4]
            p = p + irows_v[r, pl.ds(16, 16)] * wv[5]
            p = p + irows_v[r, pl.ds(32, 16)] * wv[6]
            p = p + irows_v[r, pl.ds(48, 16)] * wv[7]
            for d in (1, 2, 4, 8):
                p = p + p.at[lane ^ d].get(mode="promise_in_bounds")
            acc = jnp.where(lane == rr, p, acc)
        out_v[pl.ds(g * 16, 16)] = acc + bv
        return carry

    lax.fori_loop(0, _GROUPS, group, 0)
    pltpu.sync_copy(out_v, out_hbm.at[pl.ds(base, _BPW)])


def kernel(user_idx, item_idx, user_table, item_table, W, b):
    wf = W.reshape(-1).astype(jnp.float32)
    b16 = jnp.broadcast_to(b.astype(jnp.float32), (16,))
    mesh = plsc.VectorSubcoreMesh(core_axis_name="c", subcore_axis_name="s")
    f = pl.kernel(
        _body,
        out_type=jax.ShapeDtypeStruct((_B,), jnp.float32),
        mesh=mesh,
        compiler_params=pltpu.CompilerParams(use_tc_tiling_on_sc=False),
        scratch_types=[
            pltpu.VMEM((_NCH, _ICH), jnp.int32),
            pltpu.VMEM((_NCH, _ICH), jnp.int32),
            pltpu.VMEM((_BPW, _D), jnp.float32),
            pltpu.VMEM((_BPW, _D), jnp.float32),
            pltpu.VMEM((2 * _D,), jnp.float32),
            pltpu.VMEM((16,), jnp.float32),
            pltpu.VMEM((_BPW,), jnp.float32),
            pltpu.SemaphoreType.DMA,
        ],
    )
    return f(user_idx.astype(jnp.int32), item_idx.astype(jnp.int32),
             user_table, item_table, wf, b16)
